# baseline (speedup 1.0000x reference)
import functools
import jax
import jax.numpy as jnp
from jax import lax
from jax.experimental import pallas as pl
from jax.experimental.pallas import tpu as pltpu
from jax.experimental.pallas import tpu_sc as plsc

N_NODE = 10000
D = 128
W_SELF = 0.8
W_SEEK = 0.2
N_LAYERS = 3
E = 320000

NC = 2
NS = 16
B = 80
NB = E // (NS * B)
G = 50
NCH = NB // G
N_PAD = 10240
RPT = N_PAD // NS
RCH = 8
DCH = 80


def _pass_body(x_hbm, src_hbm, dst_hbm, ew_hbm, out_hbm,
               src_v, dst_v, ew_v, rows_v, rows_w, acc, sem, sem2,
               *, scaled):
    c = lax.axis_index("c")
    s = lax.axis_index("s")
    zeros16 = jnp.zeros((16,), jnp.float32)

    def zero_row(i, cr):
        for j in range(8):
            rows_v[i, pl.ds(16 * j, 16)] = zeros16
        return cr
    lax.fori_loop(0, B, zero_row, 0)
    for k in range(RCH):
        pltpu.sync_copy(rows_v, acc.at[pl.ds(s * RPT + k * DCH, DCH), :])

    plsc.subcore_barrier()

    def _wait_rows(buf):
        pltpu.make_async_copy(x_hbm.at[pl.ds(0, B), :], buf, sem).wait()

    def _scale_buf(buf, b2):
        if scaled:
            def scale(e, carry3):
                ewc = ew_v[pl.ds(16 * (b2 * (B // 16) + e // 16), 16)]
                w = lax.gather(
                    ewc, jnp.full((16, 1), e % 16, jnp.int32),
                    dimension_numbers=lax.GatherDimensionNumbers(
                        offset_dims=(), collapsed_slice_dims=(0,),
                        start_index_map=(0,)),
                    slice_sizes=(1,),
                    mode=lax.GatherScatterMode.PROMISE_IN_BOUNDS)
                for j in range(8):
                    buf[e, pl.ds(16 * j, 16)] = buf[e, pl.ds(16 * j, 16)] * w
                return carry3
            lax.fori_loop(0, B, scale, 0)

    def _process(buf, b2):
        _scale_buf(buf, b2)
        pltpu.sync_copy(buf, acc.at[dst_v.at[b2]], add=True)

    def chunk_body(ch, carry):
        pltpu.sync_copy(src_hbm.at[c, s, ch], src_v)
        pltpu.sync_copy(dst_hbm.at[c, s, ch], dst_v)
        if scaled:
            pltpu.sync_copy(ew_hbm.at[c, s, ch], ew_v)

        pltpu.async_copy(x_hbm.at[src_v.at[0]], rows_v, sem)

        def pair(p, carry2):
            pltpu.make_async_copy(x_hbm.at[pl.ds(0, B), :], rows_v, sem).wait()
            pltpu.async_copy(x_hbm.at[src_v.at[2 * p + 1]], rows_w, sem)
            _scale_buf(rows_v, 2 * p)
            pltpu.async_copy(rows_v, acc.at[dst_v.at[2 * p]], sem2, add=True)
            pltpu.make_async_copy(x_hbm.at[pl.ds(0, B), :], rows_w, sem).wait()
            _scale_buf(rows_w, 2 * p + 1)
            pltpu.make_async_copy(x_hbm.at[pl.ds(0, B), :], rows_v,
                                  sem2).wait()
            pltpu.async_copy(x_hbm.at[src_v.at[2 * p + 2]], rows_v, sem)
            pltpu.sync_copy(rows_w, acc.at[dst_v.at[2 * p + 1]], add=True)
            return carry2
        lax.fori_loop(0, G // 2 - 1, pair, 0)

        _wait_rows(rows_v)
        pltpu.async_copy(x_hbm.at[src_v.at[G - 1]], rows_w, sem)
        _process(rows_v, G - 2)
        _wait_rows(rows_w)
        _process(rows_w, G - 1)
        return carry
    lax.fori_loop(0, NCH, chunk_body, 0)

    plsc.subcore_barrier()

    for k in range(RCH):
        r0 = s * RPT + k * DCH
        pltpu.sync_copy(acc.at[pl.ds(r0, DCH), :], rows_v)
        pltpu.sync_copy(rows_v, out_hbm.at[c, pl.ds(r0, DCH), :])


def _make_pass(scaled):
    return pl.kernel(
        functools.partial(_pass_body, scaled=scaled),
        out_type=jax.ShapeDtypeStruct((NC, N_PAD, D), jnp.float32),
        mesh=plsc.VectorSubcoreMesh(core_axis_name="c", subcore_axis_name="s",
                                    num_cores=NC, num_subcores=NS),
        scratch_types=[
            pltpu.VMEM((G, B), jnp.int32),
            pltpu.VMEM((G, B), jnp.int32),
            pltpu.VMEM((G * B,), jnp.float32),
            pltpu.VMEM((B, D), jnp.float32),
            pltpu.VMEM((B, D), jnp.float32),
            pltpu.VMEM_SHARED((N_PAD, D), jnp.float32),
            pltpu.SemaphoreType.DMA,
            pltpu.SemaphoreType.DMA,
        ],
    )


_sc_pass_plain = _make_pass(False)
_sc_pass_w = _make_pass(True)


def _r(a):
    return a.reshape(NS, NCH, G, B)


def _norm(deg):
    deg = deg.astype(jnp.float32)
    return jnp.where(deg > 0, lax.rsqrt(jnp.maximum(deg, 1.0)), 0.0)[:, None]


def kernel(user_embedding, item_embedding, edge_user_main, edge_game_main,
           edge_user_seek, edge_game_seek, edge_user_dn, edge_game_dn,
           weight_edge):
    n_mu = _norm(jnp.bincount(edge_user_main, length=N_NODE))
    n_mg = _norm(jnp.bincount(edge_game_main, length=N_NODE))
    n_su = _norm(jnp.bincount(edge_user_seek, length=N_NODE))
    n_sg = _norm(jnp.bincount(edge_game_seek, length=N_NODE))
    n_du = _norm(jnp.bincount(edge_user_dn, length=N_NODE))
    n_dg = _norm(jnp.bincount(edge_game_dn, length=N_NODE))

    we_w = weight_edge.reshape(NS, NCH, G * B)
    ones_w = jnp.ones_like(we_w)
    zeros_w = jnp.zeros_like(we_w)

    eum, egm = _r(edge_user_main), _r(edge_game_main)
    eus, egs = _r(edge_user_seek), _r(edge_game_seek)
    eud, egd = _r(edge_user_dn), _r(edge_game_dn)

    def st(a, b):
        return jnp.stack([a, b])

    def st_src(a, b):
        return jnp.stack([a, b + N_NODE])

    def xcat(xa, xb):
        return jnp.concatenate([xa, xb])

    h_user = user_embedding
    h_game = item_embedding
    src_u = st_src(egm, egs)
    dst_u = st(eum, eus)
    ew_u = st(ones_w, we_w)
    src_g = st_src(eum, eus)
    dst_g = st(egm, egs)
    ew_z = st(zeros_w, zeros_w)
    for _ in range(N_LAYERS):
        p_u = _sc_pass_w(xcat(n_mg * h_game, n_sg * h_game), src_u, dst_u,
                         ew_u)
        p_g = _sc_pass_plain(xcat(n_mu * h_user, n_su * h_user), src_g,
                             dst_g, ew_z)
        h_user = W_SELF * n_mu * p_u[0, :N_NODE] + W_SEEK * n_su * p_u[1, :N_NODE]
        h_game = W_SELF * n_mg * p_g[0, :N_NODE] + W_SEEK * n_sg * p_g[1, :N_NODE]

    p_c = _sc_pass_w(xcat(n_sg * h_game, n_su * h_user), st_src(egs, eus),
                     st(eus, egs), st(we_w, ones_w))
    hu_s_c = W_SEEK * n_su * p_c[0, :N_NODE]
    hi_s_c = W_SEEK * n_sg * p_c[1, :N_NODE]

    h1_user = user_embedding
    h1_game = item_embedding
    src_d = st_src(egd, eud)
    dst_d = st(eud, egd)
    for _ in range(N_LAYERS):
        p = _sc_pass_plain(xcat(n_dg * h1_game, n_du * h1_user), src_d,
                           dst_d, ew_z)
        h1_user = W_SELF * n_du * p[0, :N_NODE] + hu_s_c
        h1_game = W_SELF * n_dg * p[1, :N_NODE] + hi_s_c

    return (h_user, h_game, h_user, h_game, h1_user, h1_game)


# speedup vs baseline: 1.0590x; 1.0590x over previous
import functools
import jax
import jax.numpy as jnp
from jax import lax
from jax.experimental import pallas as pl
from jax.experimental.pallas import tpu as pltpu
from jax.experimental.pallas import tpu_sc as plsc

N_NODE = 10000
D = 128
W_SELF = 0.8
W_SEEK = 0.2
N_LAYERS = 3
E = 320000

NC = 2
NS = 16
B = 80
NB = E // (NS * B)
G = 50
NCH = NB // G
N_PAD = 10240
RPT = N_PAD // NS
RCH = 8
DCH = 80


def _pass_body(x_hbm, src_hbm, dst_hbm, ew_hbm, out_hbm,
               src_v, dst_v, ew_v, rows_v, rows_w, acc, sem, sem2,
               *, scaled):
    c = lax.axis_index("c")
    s = lax.axis_index("s")
    zeros16 = jnp.zeros((16,), jnp.float32)

    def zero_row(i, cr):
        for j in range(8):
            rows_v[i, pl.ds(16 * j, 16)] = zeros16
        return cr
    lax.fori_loop(0, B, zero_row, 0)
    for k in range(RCH):
        pltpu.sync_copy(rows_v, acc.at[pl.ds(s * RPT + k * DCH, DCH), :])

    plsc.subcore_barrier()

    def _wait_rows(buf):
        pltpu.make_async_copy(x_hbm.at[pl.ds(0, B), :], buf, sem).wait()

    def _scale_buf(buf, b2):
        if scaled:
            def scale(e, carry3):
                ewc = ew_v[pl.ds(16 * (b2 * (B // 16) + e // 16), 16)]
                w = lax.gather(
                    ewc, jnp.full((16, 1), e % 16, jnp.int32),
                    dimension_numbers=lax.GatherDimensionNumbers(
                        offset_dims=(), collapsed_slice_dims=(0,),
                        start_index_map=(0,)),
                    slice_sizes=(1,),
                    mode=lax.GatherScatterMode.PROMISE_IN_BOUNDS)
                for j in range(8):
                    buf[e, pl.ds(16 * j, 16)] = buf[e, pl.ds(16 * j, 16)] * w
                return carry3
            lax.fori_loop(0, B, scale, 0)

    def _process(buf, b2):
        _scale_buf(buf, b2)
        pltpu.sync_copy(buf, acc.at[dst_v.at[b2]], add=True)

    def chunk_body(ch, carry):
        pltpu.sync_copy(src_hbm.at[c, s, ch], src_v)
        pltpu.sync_copy(dst_hbm.at[c, s, ch], dst_v)
        if scaled:
            pltpu.sync_copy(ew_hbm.at[c, s, ch], ew_v)

        pltpu.async_copy(x_hbm.at[src_v.at[0]], rows_v, sem)

        def pair(p, carry2):
            pltpu.make_async_copy(x_hbm.at[pl.ds(0, B), :], rows_v, sem).wait()
            pltpu.async_copy(x_hbm.at[src_v.at[2 * p + 1]], rows_w, sem)
            _process(rows_v, 2 * p)
            pltpu.make_async_copy(x_hbm.at[pl.ds(0, B), :], rows_w, sem).wait()
            pltpu.async_copy(x_hbm.at[src_v.at[2 * p + 2]], rows_v, sem)
            _process(rows_w, 2 * p + 1)
            return carry2
        lax.fori_loop(0, G // 2 - 1, pair, 0)

        _wait_rows(rows_v)
        pltpu.async_copy(x_hbm.at[src_v.at[G - 1]], rows_w, sem)
        _process(rows_v, G - 2)
        _wait_rows(rows_w)
        _process(rows_w, G - 1)
        return carry
    lax.fori_loop(0, NCH, chunk_body, 0)

    plsc.subcore_barrier()

    for k in range(RCH):
        r0 = s * RPT + k * DCH
        pltpu.sync_copy(acc.at[pl.ds(r0, DCH), :], rows_v)
        pltpu.sync_copy(rows_v, out_hbm.at[c, pl.ds(r0, DCH), :])


def _make_pass(scaled):
    return pl.kernel(
        functools.partial(_pass_body, scaled=scaled),
        out_type=jax.ShapeDtypeStruct((NC, N_PAD, D), jnp.float32),
        mesh=plsc.VectorSubcoreMesh(core_axis_name="c", subcore_axis_name="s",
                                    num_cores=NC, num_subcores=NS),
        scratch_types=[
            pltpu.VMEM((G, B), jnp.int32),
            pltpu.VMEM((G, B), jnp.int32),
            pltpu.VMEM((G * B,), jnp.float32),
            pltpu.VMEM((B, D), jnp.float32),
            pltpu.VMEM((B, D), jnp.float32),
            pltpu.VMEM_SHARED((N_PAD, D), jnp.float32),
            pltpu.SemaphoreType.DMA,
            pltpu.SemaphoreType.DMA,
        ],
    )


_sc_pass_plain = _make_pass(False)
_sc_pass_w = _make_pass(True)


def _r(a):
    return a.reshape(NS, NCH, G, B)


def _norm(deg):
    deg = deg.astype(jnp.float32)
    return jnp.where(deg > 0, lax.rsqrt(jnp.maximum(deg, 1.0)), 0.0)[:, None]


def kernel(user_embedding, item_embedding, edge_user_main, edge_game_main,
           edge_user_seek, edge_game_seek, edge_user_dn, edge_game_dn,
           weight_edge):
    n_mu = _norm(jnp.bincount(edge_user_main, length=N_NODE))
    n_mg = _norm(jnp.bincount(edge_game_main, length=N_NODE))
    n_su = _norm(jnp.bincount(edge_user_seek, length=N_NODE))
    n_sg = _norm(jnp.bincount(edge_game_seek, length=N_NODE))
    n_du = _norm(jnp.bincount(edge_user_dn, length=N_NODE))
    n_dg = _norm(jnp.bincount(edge_game_dn, length=N_NODE))

    we_w = weight_edge.reshape(NS, NCH, G * B)
    ones_w = jnp.ones_like(we_w)
    zeros_w = jnp.zeros_like(we_w)

    eum, egm = _r(edge_user_main), _r(edge_game_main)
    eus, egs = _r(edge_user_seek), _r(edge_game_seek)
    eud, egd = _r(edge_user_dn), _r(edge_game_dn)

    def st(a, b):
        return jnp.stack([a, b])

    def st_src(a, b):
        return jnp.stack([a, b + N_NODE])

    def xcat(xa, xb):
        return jnp.concatenate([xa, xb])

    h_user = user_embedding
    h_game = item_embedding
    src_u = st_src(egm, egs)
    dst_u = st(eum, eus)
    ew_u = st(ones_w, we_w)
    src_g = st_src(eum, eus)
    dst_g = st(egm, egs)
    ew_z = st(zeros_w, zeros_w)
    for _ in range(N_LAYERS):
        p_u = _sc_pass_w(xcat(n_mg * h_game, n_sg * h_game), src_u, dst_u,
                         ew_u)
        p_g = _sc_pass_plain(xcat(n_mu * h_user, n_su * h_user), src_g,
                             dst_g, ew_z)
        h_user = W_SELF * n_mu * p_u[0, :N_NODE] + W_SEEK * n_su * p_u[1, :N_NODE]
        h_game = W_SELF * n_mg * p_g[0, :N_NODE] + W_SEEK * n_sg * p_g[1, :N_NODE]

    p_c = _sc_pass_w(xcat(n_sg * h_game, n_su * h_user), st_src(egs, eus),
                     st(eus, egs), st(we_w, ones_w))
    hu_s_c = W_SEEK * n_su * p_c[0, :N_NODE]
    hi_s_c = W_SEEK * n_sg * p_c[1, :N_NODE]

    h1_user = user_embedding
    h1_game = item_embedding
    src_d = st_src(egd, eud)
    dst_d = st(eud, egd)
    for _ in range(N_LAYERS):
        p = _sc_pass_plain(xcat(n_dg * h1_game, n_du * h1_user), src_d,
                           dst_d, ew_z)
        h1_user = W_SELF * n_du * p[0, :N_NODE] + hu_s_c
        h1_game = W_SELF * n_dg * p[1, :N_NODE] + hi_s_c

    return (h_user, h_game, h_user, h_game, h1_user, h1_game)


# speedup vs baseline: 1.1733x; 1.1079x over previous
import functools
import jax
import jax.numpy as jnp
from jax import lax
from jax.experimental import pallas as pl
from jax.experimental.pallas import tpu as pltpu
from jax.experimental.pallas import tpu_sc as plsc

N_NODE = 10000
D = 128
W_SELF = 0.8
W_SEEK = 0.2
N_LAYERS = 3
E = 320000

NC = 2
NS = 16
N_PAD = 10240
RPT = N_PAD // NS
RCH = 8
DCH = 80

B_P, G_P = 125, 40
B_W, G_W = 80, 50


def _pass_body(x_hbm, src_hbm, dst_hbm, ew_hbm, out_hbm,
               src_v, dst_v, ew_v, rows_v, rows_w, acc, sem,
               *, scaled, bsz, g, nch):
    c = lax.axis_index("c")
    s = lax.axis_index("s")
    zeros16 = jnp.zeros((16,), jnp.float32)

    def zero_row(i, cr):
        for j in range(8):
            rows_v[i, pl.ds(16 * j, 16)] = zeros16
        return cr
    lax.fori_loop(0, DCH, zero_row, 0)
    for k in range(RCH):
        pltpu.sync_copy(rows_v.at[pl.ds(0, DCH), :],
                        acc.at[pl.ds(s * RPT + k * DCH, DCH), :])

    plsc.subcore_barrier()

    def _wait_rows(buf):
        pltpu.make_async_copy(x_hbm.at[src_v.at[0]], buf, sem).wait()

    def _scale_buf(buf, b2):
        if scaled:
            def scale(e, carry3):
                ewc = ew_v[pl.ds(16 * (b2 * (bsz // 16) + e // 16), 16)]
                w = lax.gather(
                    ewc, jnp.full((16, 1), e % 16, jnp.int32),
                    dimension_numbers=lax.GatherDimensionNumbers(
                        offset_dims=(), collapsed_slice_dims=(0,),
                        start_index_map=(0,)),
                    slice_sizes=(1,),
                    mode=lax.GatherScatterMode.PROMISE_IN_BOUNDS)
                for j in range(8):
                    buf[e, pl.ds(16 * j, 16)] = buf[e, pl.ds(16 * j, 16)] * w
                return carry3
            lax.fori_loop(0, bsz, scale, 0)

    def _process(buf, b2):
        _scale_buf(buf, b2)
        pltpu.sync_copy(buf, acc.at[dst_v.at[b2]], add=True)

    def chunk_body(ch, carry):
        pltpu.sync_copy(src_hbm.at[c, s, ch], src_v)
        pltpu.sync_copy(dst_hbm.at[c, s, ch], dst_v)
        if scaled:
            pltpu.sync_copy(ew_hbm.at[c, s, ch], ew_v)

        pltpu.async_copy(x_hbm.at[src_v.at[0]], rows_v, sem)

        def pair(p, carry2):
            _wait_rows(rows_v)
            pltpu.async_copy(x_hbm.at[src_v.at[2 * p + 1]], rows_w, sem)
            _process(rows_v, 2 * p)
            _wait_rows(rows_w)
            pltpu.async_copy(x_hbm.at[src_v.at[2 * p + 2]], rows_v, sem)
            _process(rows_w, 2 * p + 1)
            return carry2
        lax.fori_loop(0, g // 2 - 1, pair, 0)

        _wait_rows(rows_v)
        pltpu.async_copy(x_hbm.at[src_v.at[g - 1]], rows_w, sem)
        _process(rows_v, g - 2)
        _wait_rows(rows_w)
        _process(rows_w, g - 1)
        return carry
    lax.fori_loop(0, nch, chunk_body, 0)

    plsc.subcore_barrier()

    for k in range(RCH):
        r0 = s * RPT + k * DCH
        pltpu.sync_copy(acc.at[pl.ds(r0, DCH), :], rows_v.at[pl.ds(0, DCH), :])
        pltpu.sync_copy(rows_v.at[pl.ds(0, DCH), :],
                        out_hbm.at[c, pl.ds(r0, DCH), :])


def _make_pass(scaled):
    bsz, g = (B_W, G_W) if scaled else (B_P, G_P)
    nch = E // (NS * g * bsz)
    ew_shape = (g * bsz,) if scaled else (16,)
    return pl.kernel(
        functools.partial(_pass_body, scaled=scaled, bsz=bsz, g=g, nch=nch),
        out_type=jax.ShapeDtypeStruct((NC, N_PAD, D), jnp.float32),
        mesh=plsc.VectorSubcoreMesh(core_axis_name="c", subcore_axis_name="s",
                                    num_cores=NC, num_subcores=NS),
        scratch_types=[
            pltpu.VMEM((g, bsz), jnp.int32),
            pltpu.VMEM((g, bsz), jnp.int32),
            pltpu.VMEM(ew_shape, jnp.float32),
            pltpu.VMEM((bsz, D), jnp.float32),
            pltpu.VMEM((bsz, D), jnp.float32),
            pltpu.VMEM_SHARED((N_PAD, D), jnp.float32),
            pltpu.SemaphoreType.DMA,
        ],
    )


_sc_pass_plain = _make_pass(False)
_sc_pass_w = _make_pass(True)


def _rp(a):
    return a.reshape(NS, E // (NS * G_P * B_P), G_P, B_P)


def _rs(a):
    return a.reshape(NS, E // (NS * G_W * B_W), G_W, B_W)


def _norm(deg):
    deg = deg.astype(jnp.float32)
    return jnp.where(deg > 0, lax.rsqrt(jnp.maximum(deg, 1.0)), 0.0)[:, None]


def kernel(user_embedding, item_embedding, edge_user_main, edge_game_main,
           edge_user_seek, edge_game_seek, edge_user_dn, edge_game_dn,
           weight_edge):
    n_mu = _norm(jnp.bincount(edge_user_main, length=N_NODE))
    n_mg = _norm(jnp.bincount(edge_game_main, length=N_NODE))
    n_su = _norm(jnp.bincount(edge_user_seek, length=N_NODE))
    n_sg = _norm(jnp.bincount(edge_game_seek, length=N_NODE))
    n_du = _norm(jnp.bincount(edge_user_dn, length=N_NODE))
    n_dg = _norm(jnp.bincount(edge_game_dn, length=N_NODE))

    we_w = weight_edge.reshape(NS, E // (NS * G_W * B_W), G_W * B_W)
    ones_w = jnp.ones_like(we_w)
    ew_dummy = jnp.zeros((1, 16), jnp.float32)

    def st(a, b):
        return jnp.stack([a, b])

    def st_src(a, b):
        return jnp.stack([a, b + N_NODE])

    def xcat(xa, xb):
        return jnp.concatenate([xa, xb])

    h_user = user_embedding
    h_game = item_embedding
    src_u = st_src(_rs(edge_game_main), _rs(edge_game_seek))
    dst_u = st(_rs(edge_user_main), _rs(edge_user_seek))
    ew_u = st(ones_w, we_w)
    src_g = st_src(_rp(edge_user_main), _rp(edge_user_seek))
    dst_g = st(_rp(edge_game_main), _rp(edge_game_seek))
    for _ in range(N_LAYERS):
        p_u = _sc_pass_w(xcat(n_mg * h_game, n_sg * h_game), src_u, dst_u,
                         ew_u)
        p_g = _sc_pass_plain(xcat(n_mu * h_user, n_su * h_user), src_g,
                             dst_g, ew_dummy)
        h_user = W_SELF * n_mu * p_u[0, :N_NODE] + W_SEEK * n_su * p_u[1, :N_NODE]
        h_game = W_SELF * n_mg * p_g[0, :N_NODE] + W_SEEK * n_sg * p_g[1, :N_NODE]

    p_c = _sc_pass_w(xcat(n_sg * h_game, n_su * h_user),
                     st_src(_rs(edge_game_seek), _rs(edge_user_seek)),
                     st(_rs(edge_user_seek), _rs(edge_game_seek)),
                     st(we_w, ones_w))
    hu_s_c = W_SEEK * n_su * p_c[0, :N_NODE]
    hi_s_c = W_SEEK * n_sg * p_c[1, :N_NODE]

    h1_user = user_embedding
    h1_game = item_embedding
    src_d = st_src(_rp(edge_game_dn), _rp(edge_user_dn))
    dst_d = st(_rp(edge_user_dn), _rp(edge_game_dn))
    for _ in range(N_LAYERS):
        p = _sc_pass_plain(xcat(n_dg * h1_game, n_du * h1_user), src_d,
                           dst_d, ew_dummy)
        h1_user = W_SELF * n_du * p[0, :N_NODE] + hu_s_c
        h1_game = W_SELF * n_dg * p[1, :N_NODE] + hi_s_c

    return (h_user, h_game, h_user, h_game, h1_user, h1_game)
